# baseline (speedup 1.0000x reference)
import functools

import jax
import jax.numpy as jnp
from jax import lax
from jax.experimental import pallas as pl
from jax.experimental.pallas import tpu as pltpu
from jax.experimental.pallas import tpu_sc as plsc

N = 10000
NPAD = 10240
E = 320000
NC, NS = 2, 16
CHW = 625
K = 16
PH = 4
K2 = K // PH
ROWS_PER_TILE = NPAD // NS
ZR = 64


def _sc_mesh():
    return plsc.VectorSubcoreMesh(
        core_axis_name="c", subcore_axis_name="s", num_cores=NC, num_subcores=NS)


def _make_deg_kernel():
    @functools.partial(
        pl.kernel,
        out_type=jax.ShapeDtypeStruct((NC, NPAD, 8), jnp.float32),
        mesh=_sc_mesh(),
        compiler_params=pltpu.CompilerParams(use_tc_tiling_on_sc=False),
        scratch_types=[
            pltpu.VMEM((K, CHW), jnp.int32),
            pltpu.VMEM((CHW, 8), jnp.float32),
            pltpu.VMEM((ZR, 8), jnp.float32),
            pltpu.VMEM_SHARED((NPAD, 8), jnp.float32),
        ],
    )
    def deg_kernel(ei_hbm, ones_hbm, zz_hbm, out_hbm, dst_v, ones_v, stage_v, acc_sh):
        c = lax.axis_index("c")
        s = lax.axis_index("s")
        wid = c * NS + s
        pltpu.sync_copy(zz_hbm, stage_v)
        def zbody(i, carry):
            pltpu.sync_copy(stage_v, acc_sh.at[pl.ds(s * ROWS_PER_TILE + i * ZR, ZR)])
            return carry
        lax.fori_loop(0, ROWS_PER_TILE // ZR, zbody, 0)
        pltpu.sync_copy(ones_hbm, ones_v)
        pltpu.sync_copy(ei_hbm.at[1, wid], dst_v)
        plsc.subcore_barrier()
        def ebody(j, carry):
            pltpu.sync_copy(ones_v, acc_sh.at[dst_v.at[j]], add=True)
            return carry
        lax.fori_loop(0, K, ebody, 0)
        plsc.subcore_barrier()
        def obody(i, carry):
            r0 = s * ROWS_PER_TILE + i * ZR
            pltpu.sync_copy(acc_sh.at[pl.ds(r0, ZR)], stage_v)
            pltpu.sync_copy(stage_v, out_hbm.at[c, pl.ds(r0, ZR)])
            return carry
        lax.fori_loop(0, ROWS_PER_TILE // ZR, obody, 0)

    return deg_kernel


def _make_agg_kernel(d, dtype):
    @functools.partial(
        pl.kernel,
        out_type=jax.ShapeDtypeStruct((NC, NPAD, d), dtype),
        mesh=_sc_mesh(),
        compiler_params=pltpu.CompilerParams(use_tc_tiling_on_sc=False),
        scratch_types=[
            pltpu.VMEM((K2, CHW), jnp.int32),
            pltpu.VMEM((K2, CHW), jnp.int32),
            pltpu.VMEM((CHW, d), dtype),
            pltpu.VMEM((CHW, d), dtype),
            pltpu.VMEM((ZR, d), dtype),
            pltpu.VMEM_SHARED((NPAD, d), dtype),
            pltpu.SemaphoreType.DMA,
            pltpu.SemaphoreType.DMA,
        ],
    )
    def agg_kernel(ei_hbm, h_hbm, zz_hbm, out_hbm,
                   src_v, dst_v, rows_a, rows_b, stage_v, acc_sh, sem_a, sem_b):
        c = lax.axis_index("c")
        s = lax.axis_index("s")
        wid = c * NS + s
        pltpu.sync_copy(zz_hbm, stage_v)
        def zbody(i, carry):
            pltpu.sync_copy(stage_v, acc_sh.at[pl.ds(s * ROWS_PER_TILE + i * ZR, ZR)])
            return carry
        lax.fori_loop(0, ROWS_PER_TILE // ZR, zbody, 0)
        plsc.subcore_barrier()
        for p in range(PH):
            pltpu.sync_copy(ei_hbm.at[0, wid, pl.ds(p * K2, K2)], src_v)
            pltpu.sync_copy(ei_hbm.at[1, wid, pl.ds(p * K2, K2)], dst_v)
            pltpu.async_copy(h_hbm.at[src_v.at[0]], rows_a, sem_a)
            def ebody(i, carry):
                c0 = 2 * i
                c1 = 2 * i + 1
                pltpu.async_copy(h_hbm.at[src_v.at[c1]], rows_b, sem_b)
                pltpu.make_async_copy(h_hbm.at[src_v.at[c0]], rows_a, sem_a).wait()
                pltpu.sync_copy(rows_a, acc_sh.at[dst_v.at[c0]], add=True)
                @pl.when(c0 + 2 < K2)
                def _prefetch():
                    pltpu.async_copy(h_hbm.at[src_v.at[c0 + 2]], rows_a, sem_a)
                pltpu.make_async_copy(h_hbm.at[src_v.at[c1]], rows_b, sem_b).wait()
                pltpu.sync_copy(rows_b, acc_sh.at[dst_v.at[c1]], add=True)
                return carry
            lax.fori_loop(0, K2 // 2, ebody, 0)
        plsc.subcore_barrier()
        def obody(i, carry):
            r0 = s * ROWS_PER_TILE + i * ZR
            pltpu.sync_copy(acc_sh.at[pl.ds(r0, ZR)], stage_v)
            pltpu.sync_copy(stage_v, out_hbm.at[c, pl.ds(r0, ZR)])
            return carry
        lax.fori_loop(0, ROWS_PER_TILE // ZR, obody, 0)

    return agg_kernel


_BLK = 2000
_GRID = N // _BLK


def _dis_from(dg):
    deg = dg[0, :, 0] + dg[1, :, 0] + 1.0
    return lax.rsqrt(deg)


def _mm_scale_body(x_ref, w_ref, dg_ref, o_ref):
    dis = _dis_from(dg_ref[...])
    h = jnp.dot(x_ref[...], w_ref[...], preferred_element_type=jnp.float32)
    o_ref[...] = (h * dis[:, None]).astype(o_ref.dtype)


def _mid_body(agg_ref, hp_ref, dg_ref, w2_ref, b1_ref, o_ref):
    dis = _dis_from(dg_ref[...])
    agg = agg_ref[...].astype(jnp.float32)
    hp = hp_ref[...].astype(jnp.float32)
    h1 = dis[:, None] * (agg[0] + agg[1] + hp) + b1_ref[...]
    h1 = jnp.maximum(h1, 0.0)
    h2 = jnp.dot(h1, w2_ref[...], preferred_element_type=jnp.float32) * dis[:, None]
    o_ref[...] = h2.astype(o_ref.dtype)


def _fin_body(agg_ref, hp_ref, dg_ref, b2_ref, o_ref):
    dis = _dis_from(dg_ref[...])
    agg = agg_ref[...].astype(jnp.float32)
    hp = hp_ref[...].astype(jnp.float32)
    o_ref[...] = dis[:, None] * (agg[0] + agg[1] + hp) + b2_ref[...]


def _row_spec(d):
    return pl.BlockSpec((_BLK, d), lambda i: (i, 0))


def _dg_spec():
    return pl.BlockSpec((2, _BLK, 8), lambda i: (0, i, 0))


def _full_spec(r, cdim):
    return pl.BlockSpec((r, cdim), lambda i: (0, 0))


def _agg_spec(d):
    return pl.BlockSpec((2, _BLK, d), lambda i: (0, i, 0))


@jax.jit
def kernel(x, edge_index, W1, b1, W2, b2):
    d_hid = W1.shape[1]
    d_out = W2.shape[1]

    ei5 = edge_index.reshape(2, NC * NS, K, CHW)

    ones8 = jnp.ones((CHW, 8), jnp.float32)
    z8 = jnp.zeros((ZR, 8), jnp.float32)
    zh = jnp.zeros((ZR, d_hid), jnp.bfloat16)
    zo = jnp.zeros((ZR, d_out), jnp.bfloat16)

    dg = _make_deg_kernel()(ei5, ones8, z8)

    h1p = pl.pallas_call(
        _mm_scale_body,
        grid=(_GRID,),
        in_specs=[_row_spec(x.shape[1]), _full_spec(x.shape[1], d_hid), _dg_spec()],
        out_specs=_row_spec(d_hid),
        out_shape=jax.ShapeDtypeStruct((N, d_hid), jnp.bfloat16),
    )(x, W1, dg)

    agg1 = _make_agg_kernel(d_hid, jnp.bfloat16)(ei5, h1p, zh)

    h2p = pl.pallas_call(
        _mid_body,
        grid=(_GRID,),
        in_specs=[_agg_spec(d_hid), _row_spec(d_hid), _dg_spec(),
                  _full_spec(d_hid, d_out), _full_spec(1, d_hid)],
        out_specs=_row_spec(d_out),
        out_shape=jax.ShapeDtypeStruct((N, d_out), jnp.bfloat16),
    )(agg1, h1p, dg, W2, b1.reshape(1, d_hid))

    agg2 = _make_agg_kernel(d_out, jnp.bfloat16)(ei5, h2p, zo)

    out = pl.pallas_call(
        _fin_body,
        grid=(_GRID,),
        in_specs=[_agg_spec(d_out), _row_spec(d_out), _dg_spec(),
                  _full_spec(1, d_out)],
        out_specs=_row_spec(d_out),
        out_shape=jax.ShapeDtypeStruct((N, d_out), jnp.float32),
    )(agg2, h2p, dg, b2.reshape(1, d_out))

    return out


# speedup vs baseline: 1.0644x; 1.0644x over previous
import functools

import jax
import jax.numpy as jnp
from jax import lax
from jax.experimental import pallas as pl
from jax.experimental.pallas import tpu as pltpu
from jax.experimental.pallas import tpu_sc as plsc

N = 10000
NPAD = 10240
E = 320000
NC, NS = 2, 16
CHW = 500
K = 20
PH = 2
K2 = K // PH
ROWS_PER_TILE = NPAD // NS
ZR = 80


def _sc_mesh():
    return plsc.VectorSubcoreMesh(
        core_axis_name="c", subcore_axis_name="s", num_cores=NC, num_subcores=NS)


def _make_deg_kernel():
    @functools.partial(
        pl.kernel,
        out_type=jax.ShapeDtypeStruct((NC, NPAD, 8), jnp.float32),
        mesh=_sc_mesh(),
        compiler_params=pltpu.CompilerParams(use_tc_tiling_on_sc=False),
        scratch_types=[
            pltpu.VMEM((K, CHW), jnp.int32),
            pltpu.VMEM((CHW, 8), jnp.float32),
            pltpu.VMEM((ZR, 8), jnp.float32),
            pltpu.VMEM_SHARED((NPAD, 8), jnp.float32),
        ],
    )
    def deg_kernel(ei_hbm, ones_hbm, zz_hbm, out_hbm, dst_v, ones_v, stage_v, acc_sh):
        c = lax.axis_index("c")
        s = lax.axis_index("s")
        wid = c * NS + s
        pltpu.sync_copy(zz_hbm, stage_v)
        def zbody(i, carry):
            pltpu.sync_copy(stage_v, acc_sh.at[pl.ds(s * ROWS_PER_TILE + i * ZR, ZR)])
            return carry
        lax.fori_loop(0, ROWS_PER_TILE // ZR, zbody, 0)
        pltpu.sync_copy(ones_hbm, ones_v)
        pltpu.sync_copy(ei_hbm.at[1, wid], dst_v)
        plsc.subcore_barrier()
        def ebody(j, carry):
            pltpu.sync_copy(ones_v, acc_sh.at[dst_v.at[j]], add=True)
            return carry
        lax.fori_loop(0, K, ebody, 0)
        plsc.subcore_barrier()
        def obody(i, carry):
            r0 = s * ROWS_PER_TILE + i * ZR
            pltpu.sync_copy(acc_sh.at[pl.ds(r0, ZR)], stage_v)
            pltpu.sync_copy(stage_v, out_hbm.at[c, pl.ds(r0, ZR)])
            return carry
        lax.fori_loop(0, ROWS_PER_TILE // ZR, obody, 0)

    return deg_kernel


def _make_agg_kernel(d, dtype):
    @functools.partial(
        pl.kernel,
        out_type=jax.ShapeDtypeStruct((NC, NPAD, d), dtype),
        mesh=_sc_mesh(),
        compiler_params=pltpu.CompilerParams(use_tc_tiling_on_sc=False),
        scratch_types=[
            pltpu.VMEM((K2, CHW), jnp.int32),
            pltpu.VMEM((K2, CHW), jnp.int32),
            pltpu.VMEM((CHW, d), dtype),
            pltpu.VMEM((CHW, d), dtype),
            pltpu.VMEM((ZR, d), dtype),
            pltpu.VMEM_SHARED((NPAD, d), dtype),
            pltpu.SemaphoreType.DMA,
            pltpu.SemaphoreType.DMA,
        ],
    )
    def agg_kernel(ei_hbm, h_hbm, zz_hbm, out_hbm,
                   src_v, dst_v, rows_a, rows_b, stage_v, acc_sh, sem_a, sem_b):
        c = lax.axis_index("c")
        s = lax.axis_index("s")
        wid = c * NS + s
        pltpu.sync_copy(zz_hbm, stage_v)
        def zbody(i, carry):
            pltpu.sync_copy(stage_v, acc_sh.at[pl.ds(s * ROWS_PER_TILE + i * ZR, ZR)])
            return carry
        lax.fori_loop(0, ROWS_PER_TILE // ZR, zbody, 0)
        plsc.subcore_barrier()
        for p in range(PH):
            pltpu.sync_copy(ei_hbm.at[0, wid, pl.ds(p * K2, K2)], src_v)
            pltpu.sync_copy(ei_hbm.at[1, wid, pl.ds(p * K2, K2)], dst_v)
            pltpu.async_copy(h_hbm.at[src_v.at[0]], rows_a, sem_a)
            def ebody(i, carry):
                c0 = 2 * i
                c1 = 2 * i + 1
                pltpu.async_copy(h_hbm.at[src_v.at[c1]], rows_b, sem_b)
                pltpu.make_async_copy(h_hbm.at[src_v.at[c0]], rows_a, sem_a).wait()
                pltpu.sync_copy(rows_a, acc_sh.at[dst_v.at[c0]], add=True)
                @pl.when(c0 + 2 < K2)
                def _prefetch():
                    pltpu.async_copy(h_hbm.at[src_v.at[c0 + 2]], rows_a, sem_a)
                pltpu.make_async_copy(h_hbm.at[src_v.at[c1]], rows_b, sem_b).wait()
                pltpu.sync_copy(rows_b, acc_sh.at[dst_v.at[c1]], add=True)
                return carry
            lax.fori_loop(0, K2 // 2, ebody, 0)
        plsc.subcore_barrier()
        def obody(i, carry):
            r0 = s * ROWS_PER_TILE + i * ZR
            pltpu.sync_copy(acc_sh.at[pl.ds(r0, ZR)], stage_v)
            pltpu.sync_copy(stage_v, out_hbm.at[c, pl.ds(r0, ZR)])
            return carry
        lax.fori_loop(0, ROWS_PER_TILE // ZR, obody, 0)

    return agg_kernel


_BLK = 2000
_GRID = N // _BLK


def _dis_from(dg):
    deg = dg[0, :, 0] + dg[1, :, 0] + 1.0
    return lax.rsqrt(deg)


def _mm_scale_body(x_ref, w_ref, dg_ref, o_ref):
    dis = _dis_from(dg_ref[...])
    h = jnp.dot(x_ref[...], w_ref[...], preferred_element_type=jnp.float32)
    o_ref[...] = (h * dis[:, None]).astype(o_ref.dtype)


def _mid_body(agg_ref, hp_ref, dg_ref, w2_ref, b1_ref, o_ref):
    dis = _dis_from(dg_ref[...])
    agg = agg_ref[...].astype(jnp.float32)
    hp = hp_ref[...].astype(jnp.float32)
    h1 = dis[:, None] * (agg[0] + agg[1] + hp) + b1_ref[...]
    h1 = jnp.maximum(h1, 0.0)
    h2 = jnp.dot(h1, w2_ref[...], preferred_element_type=jnp.float32) * dis[:, None]
    o_ref[...] = h2.astype(o_ref.dtype)


def _fin_body(agg_ref, hp_ref, dg_ref, b2_ref, o_ref):
    dis = _dis_from(dg_ref[...])
    agg = agg_ref[...].astype(jnp.float32)
    hp = hp_ref[...].astype(jnp.float32)
    o_ref[...] = dis[:, None] * (agg[0] + agg[1] + hp) + b2_ref[...]


def _row_spec(d):
    return pl.BlockSpec((_BLK, d), lambda i: (i, 0))


def _dg_spec():
    return pl.BlockSpec((2, _BLK, 8), lambda i: (0, i, 0))


def _full_spec(r, cdim):
    return pl.BlockSpec((r, cdim), lambda i: (0, 0))


def _agg_spec(d):
    return pl.BlockSpec((2, _BLK, d), lambda i: (0, i, 0))


@jax.jit
def kernel(x, edge_index, W1, b1, W2, b2):
    d_hid = W1.shape[1]
    d_out = W2.shape[1]

    ei5 = edge_index.reshape(2, NC * NS, K, CHW)

    ones8 = jnp.ones((CHW, 8), jnp.float32)
    z8 = jnp.zeros((ZR, 8), jnp.float32)
    zh = jnp.zeros((ZR, d_hid), jnp.bfloat16)
    zo = jnp.zeros((ZR, d_out), jnp.bfloat16)

    dg = _make_deg_kernel()(ei5, ones8, z8)

    h1p = pl.pallas_call(
        _mm_scale_body,
        grid=(_GRID,),
        in_specs=[_row_spec(x.shape[1]), _full_spec(x.shape[1], d_hid), _dg_spec()],
        out_specs=_row_spec(d_hid),
        out_shape=jax.ShapeDtypeStruct((N, d_hid), jnp.bfloat16),
    )(x, W1, dg)

    agg1 = _make_agg_kernel(d_hid, jnp.bfloat16)(ei5, h1p, zh)

    h2p = pl.pallas_call(
        _mid_body,
        grid=(_GRID,),
        in_specs=[_agg_spec(d_hid), _row_spec(d_hid), _dg_spec(),
                  _full_spec(d_hid, d_out), _full_spec(1, d_hid)],
        out_specs=_row_spec(d_out),
        out_shape=jax.ShapeDtypeStruct((N, d_out), jnp.bfloat16),
    )(agg1, h1p, dg, W2, b1.reshape(1, d_hid))

    agg2 = _make_agg_kernel(d_out, jnp.bfloat16)(ei5, h2p, zo)

    out = pl.pallas_call(
        _fin_body,
        grid=(_GRID,),
        in_specs=[_agg_spec(d_out), _row_spec(d_out), _dg_spec(),
                  _full_spec(1, d_out)],
        out_specs=_row_spec(d_out),
        out_shape=jax.ShapeDtypeStruct((N, d_out), jnp.float32),
    )(agg2, h2p, dg, b2.reshape(1, d_out))

    return out
